# Initial kernel scaffold; baseline (speedup 1.0000x reference)
import jax
import jax.numpy as jnp
from jax.experimental import pallas as pl


def kernel(x, edge_index, W1, b1, W2, b2):
    raise NotImplementedError("write your pallas kernel here")


# speedup vs baseline: 14.4149x; 14.4149x over previous
import functools

import jax
import jax.numpy as jnp
from jax import lax
from jax.experimental import pallas as pl
from jax.experimental.pallas import tpu as pltpu
from jax.experimental.pallas import tpu_sc as plsc

N = 10000
E = 320000
F = 128
NC, NS, LANES = 2, 16, 16
NW = NC * NS
EDGES_PER_TILE = E // NW
CHUNK = 80
N_CHUNKS = EDGES_PER_TILE // CHUNK
N_PAD = 10240
DEG_PER_TILE = N_PAD // NS
ROWS_PER_TILE = N_PAD // NS
ZROWS = 128

_mesh = plsc.VectorSubcoreMesh(
    core_axis_name="c", subcore_axis_name="s", num_cores=NC, num_subcores=NS
)


@functools.partial(
    pl.kernel,
    out_type=jax.ShapeDtypeStruct((NC * N_PAD,), jnp.float32),
    mesh=_mesh,
    scratch_types=[
        pltpu.VMEM((CHUNK,), jnp.int32),
        pltpu.VMEM((CHUNK,), jnp.float32),
        pltpu.VMEM((DEG_PER_TILE,), jnp.float32),
        pltpu.VMEM_SHARED((N_PAD,), jnp.float32),
    ],
)
def _deg_kernel(dst_hbm, out_hbm, idx_v, ones_v, zbuf_v, deg_sh):
    cid = lax.axis_index("c")
    sid = lax.axis_index("s")

    zeros16 = jnp.zeros((LANES,), jnp.float32)
    for i in range(CHUNK // LANES):
        ones_v[pl.ds(i * LANES, LANES)] = zeros16 + 1.0

    @pl.loop(0, DEG_PER_TILE // LANES)
    def _(i):
        zbuf_v[pl.ds(i * LANES, LANES)] = zeros16

    pltpu.sync_copy(zbuf_v, deg_sh.at[pl.ds(sid * DEG_PER_TILE, DEG_PER_TILE)])
    plsc.subcore_barrier()

    base = (cid * NS + sid) * EDGES_PER_TILE

    @pl.loop(0, N_CHUNKS)
    def _(i):
        pltpu.sync_copy(dst_hbm.at[pl.ds(base + i * CHUNK, CHUNK)], idx_v)
        pltpu.sync_copy(ones_v, deg_sh.at[idx_v], add=True)

    plsc.subcore_barrier()
    pltpu.sync_copy(
        deg_sh.at[pl.ds(sid * DEG_PER_TILE, DEG_PER_TILE)],
        out_hbm.at[pl.ds(cid * N_PAD + sid * DEG_PER_TILE, DEG_PER_TILE)],
    )


@functools.partial(
    pl.kernel,
    out_type=jax.ShapeDtypeStruct((NC * N_PAD, F), jnp.float32),
    mesh=_mesh,
    scratch_types=[
        pltpu.VMEM((CHUNK,), jnp.int32),
        pltpu.VMEM((CHUNK,), jnp.int32),
        pltpu.VMEM((CHUNK, F), jnp.float32),
        pltpu.VMEM((ZROWS, F), jnp.float32),
        pltpu.VMEM_SHARED((N_PAD, F), jnp.float32),
        pltpu.SemaphoreType.DMA,
    ],
)
def _agg_kernel(xs_hbm, src_hbm, dst_hbm, out_hbm, src_v, dst_v, rows_v, zbuf_v,
                acc_sh, sem):
    cid = lax.axis_index("c")
    sid = lax.axis_index("s")

    zeros16 = jnp.zeros((LANES,), jnp.float32)

    @pl.loop(0, ZROWS)
    def _(r):
        for j in range(F // LANES):
            zbuf_v[r, pl.ds(j * LANES, LANES)] = zeros16

    @pl.loop(0, ROWS_PER_TILE // ZROWS)
    def _(i):
        pltpu.sync_copy(
            zbuf_v, acc_sh.at[pl.ds(sid * ROWS_PER_TILE + i * ZROWS, ZROWS)]
        )

    plsc.subcore_barrier()

    base = (cid * NS + sid) * EDGES_PER_TILE

    @pl.loop(0, N_CHUNKS)
    def _(i):
        off = base + i * CHUNK
        pltpu.sync_copy(src_hbm.at[pl.ds(off, CHUNK)], src_v)
        pltpu.sync_copy(dst_hbm.at[pl.ds(off, CHUNK)], dst_v)
        pltpu.async_copy(xs_hbm.at[src_v], rows_v, sem).wait()
        pltpu.sync_copy(rows_v, acc_sh.at[dst_v], add=True)

    plsc.subcore_barrier()
    pltpu.sync_copy(
        acc_sh.at[pl.ds(sid * ROWS_PER_TILE, ROWS_PER_TILE)],
        out_hbm.at[pl.ds(cid * N_PAD + sid * ROWS_PER_TILE, ROWS_PER_TILE)],
    )


_RB = 1000
_GRID = N // _RB

_DN = (((1,), (1,)), ((), ()))


def _scale_body(deg_ref, x_ref, o_ref):
    dinv = lax.rsqrt(deg_ref[:, 0:1] + deg_ref[:, 1:2] + 1.0)
    o_ref[...] = x_ref[...] * dinv


def _mm_body(deg_ref, p_ref, xs_ref, w1_ref, b1_ref, w2_ref, o_ref):
    dinv = lax.rsqrt(deg_ref[:, 0:1] + deg_ref[:, 1:2] + 1.0)
    agg = (p_ref[0] + p_ref[1] + xs_ref[...]) * dinv
    h = lax.dot_general(agg, w1_ref[...], _DN,
                        preferred_element_type=jnp.float32,
                        precision=lax.Precision.HIGHEST)
    h = jnp.maximum(h + b1_ref[...], 0.0)
    hw = lax.dot_general(h, w2_ref[...], _DN,
                         preferred_element_type=jnp.float32,
                         precision=lax.Precision.HIGHEST)
    o_ref[...] = hw * dinv


def _fin_body(deg_ref, q_ref, hws_ref, b2_ref, o_ref):
    dinv = lax.rsqrt(deg_ref[:, 0:1] + deg_ref[:, 1:2] + 1.0)
    z = (q_ref[0] + q_ref[1] + hws_ref[...]) * dinv + b2_ref[...]
    m = jnp.max(z, axis=1, keepdims=True)
    e = jnp.exp(z - m)
    s = jnp.sum(e, axis=1, keepdims=True)
    o_ref[...] = z - m - jnp.log(s)


_deg_spec = pl.BlockSpec((_RB, NC), lambda i: (i, 0))
_row_spec = pl.BlockSpec((_RB, F), lambda i: (i, 0))
_pair_spec = pl.BlockSpec((NC, _RB, F), lambda i: (0, i, 0))

_scale_call = pl.pallas_call(
    _scale_body,
    out_shape=jax.ShapeDtypeStruct((N, F), jnp.float32),
    grid=(_GRID,),
    in_specs=[_deg_spec, _row_spec],
    out_specs=_row_spec,
)

_mm_call = pl.pallas_call(
    _mm_body,
    out_shape=jax.ShapeDtypeStruct((N, F), jnp.float32),
    grid=(_GRID,),
    in_specs=[
        _deg_spec,
        _pair_spec,
        _row_spec,
        pl.BlockSpec((256, F), lambda i: (0, 0)),
        pl.BlockSpec((1, 256), lambda i: (0, 0)),
        pl.BlockSpec((F, 256), lambda i: (0, 0)),
    ],
    out_specs=_row_spec,
)

_fin_call = pl.pallas_call(
    _fin_body,
    out_shape=jax.ShapeDtypeStruct((N, F), jnp.float32),
    grid=(_GRID,),
    in_specs=[
        _deg_spec,
        _pair_spec,
        _row_spec,
        pl.BlockSpec((1, F), lambda i: (0, 0)),
    ],
    out_specs=_row_spec,
)


def kernel(x, edge_index, W1, b1, W2, b2):
    src = edge_index[0]
    dst = edge_index[1]

    deg_flat = _deg_kernel(dst)
    degT = deg_flat.reshape(NC, N_PAD).T

    xs = _scale_call(degT, x)
    P = _agg_kernel(xs, src, dst).reshape(NC, N_PAD, F)
    hws = _mm_call(degT, P, xs, W1, b1.reshape(1, -1), W2)
    Q = _agg_kernel(hws, src, dst).reshape(NC, N_PAD, F)
    return _fin_call(degT, Q, hws, b2.reshape(1, -1))


# speedup vs baseline: 27.5198x; 1.9091x over previous
import functools

import jax
import jax.numpy as jnp
from jax import lax
from jax.experimental import pallas as pl
from jax.experimental.pallas import tpu as pltpu
from jax.experimental.pallas import tpu_sc as plsc

N = 10000
E = 320000
F = 128
NC, NS, LANES = 2, 16, 16
NW = NC * NS
EDGES_PER_TILE = E // NW
CHUNK = 80
N_CHUNKS = EDGES_PER_TILE // CHUNK
NBUF = 5
ABUF = 2
N_PAD = 10240
DEG_PER_TILE = N_PAD // NS
ROWS_PER_TILE = N_PAD // NS
ZROWS = 128

_mesh = plsc.VectorSubcoreMesh(
    core_axis_name="c", subcore_axis_name="s", num_cores=NC, num_subcores=NS
)


@functools.partial(
    pl.kernel,
    out_type=jax.ShapeDtypeStruct((NC * N_PAD,), jnp.float32),
    mesh=_mesh,
    scratch_types=[
        pltpu.VMEM((N_CHUNKS, CHUNK), jnp.int32),
        pltpu.VMEM((CHUNK,), jnp.float32),
        pltpu.VMEM((DEG_PER_TILE,), jnp.float32),
        pltpu.VMEM_SHARED((N_PAD,), jnp.float32),
        pltpu.SemaphoreType.DMA,
    ] + [pltpu.SemaphoreType.DMA] * NBUF,
)
def _deg_kernel(dst_hbm, out_hbm, idx_v, ones_v, zbuf_v, deg_sh, isem, *ssem):
    cid = lax.axis_index("c")
    sid = lax.axis_index("s")
    wid = cid * NS + sid

    idx_cp = pltpu.async_copy(dst_hbm.at[wid], idx_v, isem)

    zeros16 = jnp.zeros((LANES,), jnp.float32)
    for i in range(CHUNK // LANES):
        ones_v[pl.ds(i * LANES, LANES)] = zeros16 + 1.0

    @pl.loop(0, DEG_PER_TILE // LANES)
    def _(i):
        zbuf_v[pl.ds(i * LANES, LANES)] = zeros16

    pltpu.sync_copy(zbuf_v, deg_sh.at[pl.ds(sid * DEG_PER_TILE, DEG_PER_TILE)])
    plsc.subcore_barrier()
    idx_cp.wait()

    def _fire(k, b):
        pltpu.async_copy(ones_v, deg_sh.at[idx_v.at[k]], ssem[b], add=True)

    def _drain(b):
        pltpu.make_async_copy(ones_v, deg_sh.at[idx_v.at[0]], ssem[b]).wait()

    for b in range(NBUF):
        _fire(b, b)

    @pl.loop(NBUF, N_CHUNKS, step=NBUF)
    def _(i):
        for b in range(NBUF):
            _drain(b)
            _fire(i + b, b)

    for b in range(NBUF):
        _drain(b)

    plsc.subcore_barrier()
    pltpu.sync_copy(
        deg_sh.at[pl.ds(sid * DEG_PER_TILE, DEG_PER_TILE)],
        out_hbm.at[pl.ds(cid * N_PAD + sid * DEG_PER_TILE, DEG_PER_TILE)],
    )


@functools.partial(
    pl.kernel,
    out_type=jax.ShapeDtypeStruct((NC * N_PAD, F), jnp.float32),
    mesh=_mesh,
    scratch_types=[
        pltpu.VMEM((N_CHUNKS, CHUNK), jnp.int32),
        pltpu.VMEM((ABUF, CHUNK), jnp.int32),
        pltpu.VMEM((ABUF, CHUNK), jnp.int32),
        pltpu.VMEM((ABUF, CHUNK, F), jnp.float32),
        pltpu.VMEM_SHARED((N_PAD, F), jnp.float32),
        pltpu.SemaphoreType.DMA,
    ] + [pltpu.SemaphoreType.DMA] * (2 * ABUF),
)
def _agg_kernel(xs_hbm, pk_hbm, out_hbm, pk_v, src_v, dst_v, rows_v,
                acc_sh, isem0, *sems):
    gsem = sems[:ABUF]
    ssem = sems[ABUF:]
    cid = lax.axis_index("c")
    sid = lax.axis_index("s")
    wid = cid * NS + sid

    pk_cp = pltpu.async_copy(pk_hbm.at[wid], pk_v, isem0)

    zeros16 = jnp.zeros((LANES,), jnp.float32)

    @pl.loop(0, CHUNK)
    def _(r):
        for j in range(F // LANES):
            rows_v[0, r, pl.ds(j * LANES, LANES)] = zeros16

    @pl.loop(0, ROWS_PER_TILE // CHUNK)
    def _(i):
        pltpu.sync_copy(
            rows_v.at[0], acc_sh.at[pl.ds(sid * ROWS_PER_TILE + i * CHUNK, CHUNK)]
        )

    plsc.subcore_barrier()
    pk_cp.wait()

    mask16 = jnp.full((LANES,), 0xFFFF, jnp.int32)
    sh16 = jnp.full((LANES,), 16, jnp.int32)

    def _gather(k, b):
        for j in range(CHUNK // LANES):
            w = pk_v[k, pl.ds(j * LANES, LANES)]
            src_v[b, pl.ds(j * LANES, LANES)] = lax.bitwise_and(w, mask16)
            dst_v[b, pl.ds(j * LANES, LANES)] = lax.shift_right_logical(w, sh16)
        pltpu.async_copy(xs_hbm.at[src_v.at[b]], rows_v.at[b], gsem[b])

    def _gwait(b):
        pltpu.make_async_copy(xs_hbm.at[src_v.at[b]], rows_v.at[b],
                              gsem[b]).wait()

    def _scatter(k, b):
        pltpu.async_copy(rows_v.at[b], acc_sh.at[dst_v.at[b]], ssem[b],
                         add=True)

    def _swait(b):
        pltpu.make_async_copy(rows_v.at[b], acc_sh.at[dst_v.at[b]],
                              ssem[b]).wait()

    for b in range(ABUF):
        _gather(b, b)

    @pl.loop(0, N_CHUNKS, step=ABUF)
    def _(i):
        for b in range(ABUF):
            @pl.when(i + b < N_CHUNKS)
            def _():
                _gwait(b)
                _scatter(i + b, b)
        for b in range(ABUF):
            nk = i + b + ABUF

            @pl.when(nk < N_CHUNKS)
            def _():
                _swait(b)
                _gather(nk, b)

    for b in range(ABUF):
        _swait(b)

    plsc.subcore_barrier()
    pltpu.sync_copy(
        acc_sh.at[pl.ds(sid * ROWS_PER_TILE, ROWS_PER_TILE)],
        out_hbm.at[pl.ds(cid * N_PAD + sid * ROWS_PER_TILE, ROWS_PER_TILE)],
    )


_RB = 1000
_GRID = N // _RB

_DN = (((1,), (1,)), ((), ()))


def _scale_body(deg_ref, x_ref, o_ref):
    dinv = lax.rsqrt(deg_ref[:, 0:1] + deg_ref[:, 1:2] + 1.0)
    o_ref[...] = x_ref[...] * dinv


def _mm_body(deg_ref, p_ref, xs_ref, w1_ref, b1_ref, w2_ref, o_ref):
    dinv = lax.rsqrt(deg_ref[:, 0:1] + deg_ref[:, 1:2] + 1.0)
    agg = (p_ref[0] + p_ref[1] + xs_ref[...]) * dinv
    h = lax.dot_general(agg, w1_ref[...], _DN,
                        preferred_element_type=jnp.float32,
                        precision=lax.Precision.HIGHEST)
    h = jnp.maximum(h + b1_ref[...], 0.0)
    hw = lax.dot_general(h, w2_ref[...], _DN,
                         preferred_element_type=jnp.float32,
                         precision=lax.Precision.HIGHEST)
    o_ref[...] = hw * dinv


def _fin_body(deg_ref, q_ref, hws_ref, b2_ref, o_ref):
    dinv = lax.rsqrt(deg_ref[:, 0:1] + deg_ref[:, 1:2] + 1.0)
    z = (q_ref[0] + q_ref[1] + hws_ref[...]) * dinv + b2_ref[...]
    m = jnp.max(z, axis=1, keepdims=True)
    e = jnp.exp(z - m)
    s = jnp.sum(e, axis=1, keepdims=True)
    o_ref[...] = z - m - jnp.log(s)


_deg_spec = pl.BlockSpec((_RB, NC), lambda i: (i, 0))
_row_spec = pl.BlockSpec((_RB, F), lambda i: (i, 0))
_pair_spec = pl.BlockSpec((NC, _RB, F), lambda i: (0, i, 0))

_scale_call = pl.pallas_call(
    _scale_body,
    out_shape=jax.ShapeDtypeStruct((N, F), jnp.float32),
    grid=(_GRID,),
    in_specs=[_deg_spec, _row_spec],
    out_specs=_row_spec,
)

_mm_call = pl.pallas_call(
    _mm_body,
    out_shape=jax.ShapeDtypeStruct((N, F), jnp.float32),
    grid=(_GRID,),
    in_specs=[
        _deg_spec,
        _pair_spec,
        _row_spec,
        pl.BlockSpec((256, F), lambda i: (0, 0)),
        pl.BlockSpec((1, 256), lambda i: (0, 0)),
        pl.BlockSpec((F, 256), lambda i: (0, 0)),
    ],
    out_specs=_row_spec,
)

_fin_call = pl.pallas_call(
    _fin_body,
    out_shape=jax.ShapeDtypeStruct((N, F), jnp.float32),
    grid=(_GRID,),
    in_specs=[
        _deg_spec,
        _pair_spec,
        _row_spec,
        pl.BlockSpec((1, F), lambda i: (0, 0)),
    ],
    out_specs=_row_spec,
)


def kernel(x, edge_index, W1, b1, W2, b2):
    src = edge_index[0]
    dst = edge_index[1]
    packed = (
        lax.shift_left(dst, 16) | src
    ).reshape(NW, N_CHUNKS, CHUNK)
    dst3 = dst.reshape(NW, N_CHUNKS, CHUNK)

    deg_flat = _deg_kernel(dst3)
    degT = deg_flat.reshape(NC, N_PAD).T

    xs = _scale_call(degT, x)
    P = _agg_kernel(xs, packed).reshape(NC, N_PAD, F)
    hws = _mm_call(degT, P, xs, W1, b1.reshape(1, -1), W2)
    Q = _agg_kernel(hws, packed).reshape(NC, N_PAD, F)
    return _fin_call(degT, Q, hws, b2.reshape(1, -1))


# speedup vs baseline: 32.9284x; 1.1965x over previous
import functools

import jax
import jax.numpy as jnp
from jax import lax
from jax.experimental import pallas as pl
from jax.experimental.pallas import tpu as pltpu
from jax.experimental.pallas import tpu_sc as plsc

N = 10000
E = 320000
F = 128
NC, NS, LANES = 2, 16, 16
NW = NC * NS
EDGES_PER_TILE = E // NW
CHUNK = 80
N_CHUNKS = EDGES_PER_TILE // CHUNK
DBUF = 5
ABUF = 3
N_PAD = 10240
DEG_PER_TILE = N_PAD // NS
ROWS_PER_TILE = N_PAD // NS

_mesh = plsc.VectorSubcoreMesh(
    core_axis_name="c", subcore_axis_name="s", num_cores=NC, num_subcores=NS
)


@functools.partial(
    pl.kernel,
    out_type=jax.ShapeDtypeStruct((NC * N_PAD,), jnp.float32),
    mesh=_mesh,
    scratch_types=[
        pltpu.VMEM((N_CHUNKS, CHUNK), jnp.int32),
        pltpu.VMEM((CHUNK,), jnp.float32),
        pltpu.VMEM((DEG_PER_TILE,), jnp.float32),
        pltpu.VMEM_SHARED((N_PAD,), jnp.float32),
        pltpu.SemaphoreType.DMA,
    ] + [pltpu.SemaphoreType.DMA] * DBUF,
)
def _deg_kernel(dst_hbm, out_hbm, idx_v, ones_v, zbuf_v, deg_sh, isem, *ssem):
    cid = lax.axis_index("c")
    sid = lax.axis_index("s")
    wid = cid * NS + sid

    idx_cp = pltpu.async_copy(dst_hbm.at[wid], idx_v, isem)

    zeros16 = jnp.zeros((LANES,), jnp.float32)
    for i in range(CHUNK // LANES):
        ones_v[pl.ds(i * LANES, LANES)] = zeros16 + 1.0

    @pl.loop(0, DEG_PER_TILE // LANES)
    def _(i):
        zbuf_v[pl.ds(i * LANES, LANES)] = zeros16

    pltpu.sync_copy(zbuf_v, deg_sh.at[pl.ds(sid * DEG_PER_TILE, DEG_PER_TILE)])
    plsc.subcore_barrier()
    idx_cp.wait()

    def _fire(k, b):
        pltpu.async_copy(ones_v, deg_sh.at[idx_v.at[k]], ssem[b], add=True)

    def _drain(b):
        pltpu.make_async_copy(ones_v, deg_sh.at[idx_v.at[0]], ssem[b]).wait()

    for b in range(DBUF):
        _fire(b, b)

    @pl.loop(DBUF, N_CHUNKS, step=DBUF)
    def _(i):
        for b in range(DBUF):
            _drain(b)
            _fire(i + b, b)

    for b in range(DBUF):
        _drain(b)

    plsc.subcore_barrier()
    pltpu.sync_copy(
        deg_sh.at[pl.ds(sid * DEG_PER_TILE, DEG_PER_TILE)],
        out_hbm.at[pl.ds(cid * N_PAD + sid * DEG_PER_TILE, DEG_PER_TILE)],
    )


@functools.partial(
    pl.kernel,
    out_type=jax.ShapeDtypeStruct((NC * N_PAD, F), jnp.float32),
    mesh=_mesh,
    scratch_types=[
        pltpu.VMEM((N_CHUNKS, CHUNK), jnp.int32),
        pltpu.VMEM((ABUF, CHUNK), jnp.int32),
        pltpu.VMEM((ABUF, CHUNK), jnp.int32),
        pltpu.VMEM((ABUF, CHUNK, F), jnp.float32),
        pltpu.VMEM_SHARED((N_PAD, F), jnp.float32),
        pltpu.SemaphoreType.DMA,
    ] + [pltpu.SemaphoreType.DMA] * (2 * ABUF),
)
def _agg_kernel(xs_hbm, pk_hbm, out_hbm, pk_v, src_v, dst_v, rows_v,
                acc_sh, isem0, *sems):
    gsem = sems[:ABUF]
    ssem = sems[ABUF:]
    cid = lax.axis_index("c")
    sid = lax.axis_index("s")
    wid = cid * NS + sid

    pk_cp = pltpu.async_copy(pk_hbm.at[wid], pk_v, isem0)

    zeros16 = jnp.zeros((LANES,), jnp.float32)

    @pl.loop(0, CHUNK)
    def _(r):
        for j in range(F // LANES):
            rows_v[0, r, pl.ds(j * LANES, LANES)] = zeros16

    @pl.loop(0, ROWS_PER_TILE // CHUNK)
    def _(i):
        pltpu.sync_copy(
            rows_v.at[0], acc_sh.at[pl.ds(sid * ROWS_PER_TILE + i * CHUNK, CHUNK)]
        )

    plsc.subcore_barrier()
    pk_cp.wait()

    mask16 = jnp.full((LANES,), 0xFFFF, jnp.int32)
    sh16 = jnp.full((LANES,), 16, jnp.int32)

    def _gather(k, b):
        for j in range(CHUNK // LANES):
            w = pk_v[k, pl.ds(j * LANES, LANES)]
            src_v[b, pl.ds(j * LANES, LANES)] = lax.bitwise_and(w, mask16)
            dst_v[b, pl.ds(j * LANES, LANES)] = lax.shift_right_logical(w, sh16)
        pltpu.async_copy(xs_hbm.at[src_v.at[b]], rows_v.at[b], gsem[b])

    def _gwait(b):
        pltpu.make_async_copy(xs_hbm.at[src_v.at[b]], rows_v.at[b],
                              gsem[b]).wait()

    def _scatter(b):
        pltpu.async_copy(rows_v.at[b], acc_sh.at[dst_v.at[b]], ssem[b],
                         add=True)

    def _swait(b):
        pltpu.make_async_copy(rows_v.at[b], acc_sh.at[dst_v.at[b]],
                              ssem[b]).wait()

    for b in range(ABUF):
        _gather(b, b)

    @pl.loop(0, N_CHUNKS, step=ABUF)
    def _(i):
        for b in range(ABUF):
            @pl.when(i + b < N_CHUNKS)
            def _():
                _gwait(b)
                _scatter(b)
        for b in range(ABUF):
            nk = i + b + ABUF

            @pl.when(nk < N_CHUNKS)
            def _():
                _swait(b)
                _gather(nk, b)

    for b in range(ABUF):
        _swait(b)

    plsc.subcore_barrier()
    pltpu.sync_copy(
        acc_sh.at[pl.ds(sid * ROWS_PER_TILE, ROWS_PER_TILE)],
        out_hbm.at[pl.ds(cid * N_PAD + sid * ROWS_PER_TILE, ROWS_PER_TILE)],
    )


_RB = 1000
_GRID = N // _RB

_DN = (((1,), (1,)), ((), ()))


def _scale_body(deg_ref, x_ref, o_ref):
    dinv = lax.rsqrt(deg_ref[:, 0:1] + deg_ref[:, 1:2] + 1.0)
    o_ref[...] = x_ref[...] * dinv


def _mm_body(deg_ref, p_ref, xs_ref, w1_ref, b1_ref, w2_ref, o_ref):
    dinv = lax.rsqrt(deg_ref[:, 0:1] + deg_ref[:, 1:2] + 1.0)
    agg = (p_ref[0] + p_ref[1] + xs_ref[...]) * dinv
    h = lax.dot_general(agg, w1_ref[...], _DN,
                        preferred_element_type=jnp.float32,
                        precision=lax.Precision.HIGHEST)
    h = jnp.maximum(h + b1_ref[...], 0.0)
    hw = lax.dot_general(h, w2_ref[...], _DN,
                         preferred_element_type=jnp.float32,
                         precision=lax.Precision.HIGHEST)
    o_ref[...] = hw * dinv


def _fin_body(deg_ref, q_ref, hws_ref, b2_ref, o_ref):
    dinv = lax.rsqrt(deg_ref[:, 0:1] + deg_ref[:, 1:2] + 1.0)
    z = (q_ref[0] + q_ref[1] + hws_ref[...]) * dinv + b2_ref[...]
    m = jnp.max(z, axis=1, keepdims=True)
    e = jnp.exp(z - m)
    s = jnp.sum(e, axis=1, keepdims=True)
    o_ref[...] = z - m - jnp.log(s)


_deg_spec = pl.BlockSpec((_RB, NC), lambda i: (i, 0))
_row_spec = pl.BlockSpec((_RB, F), lambda i: (i, 0))
_pair_spec = pl.BlockSpec((NC, _RB, F), lambda i: (0, i, 0))

_scale_call = pl.pallas_call(
    _scale_body,
    out_shape=jax.ShapeDtypeStruct((N, F), jnp.float32),
    grid=(_GRID,),
    in_specs=[_deg_spec, _row_spec],
    out_specs=_row_spec,
)

_mm_call = pl.pallas_call(
    _mm_body,
    out_shape=jax.ShapeDtypeStruct((N, F), jnp.float32),
    grid=(_GRID,),
    in_specs=[
        _deg_spec,
        _pair_spec,
        _row_spec,
        pl.BlockSpec((256, F), lambda i: (0, 0)),
        pl.BlockSpec((1, 256), lambda i: (0, 0)),
        pl.BlockSpec((F, 256), lambda i: (0, 0)),
    ],
    out_specs=_row_spec,
)

_fin_call = pl.pallas_call(
    _fin_body,
    out_shape=jax.ShapeDtypeStruct((N, F), jnp.float32),
    grid=(_GRID,),
    in_specs=[
        _deg_spec,
        _pair_spec,
        _row_spec,
        pl.BlockSpec((1, F), lambda i: (0, 0)),
    ],
    out_specs=_row_spec,
)


def kernel(x, edge_index, W1, b1, W2, b2):
    src = edge_index[0]
    dst = edge_index[1]
    packed = (lax.shift_left(dst, 16) | src).reshape(NW, N_CHUNKS, CHUNK)
    dst3 = dst.reshape(NW, N_CHUNKS, CHUNK)

    deg_flat = _deg_kernel(dst3)
    degT = deg_flat.reshape(NC, N_PAD).T

    xs = _scale_call(degT, x)
    P = _agg_kernel(xs, packed).reshape(NC, N_PAD, F)
    hws = _mm_call(degT, P, xs, W1, b1.reshape(1, -1), W2)
    Q = _agg_kernel(hws, packed).reshape(NC, N_PAD, F)
    return _fin_call(degT, Q, hws, b2.reshape(1, -1))


# speedup vs baseline: 35.4791x; 1.0775x over previous
import functools

import jax
import jax.numpy as jnp
from jax import lax
from jax.experimental import pallas as pl
from jax.experimental.pallas import tpu as pltpu
from jax.experimental.pallas import tpu_sc as plsc

N = 10000
E = 320000
F = 128
NC, NS, LANES = 2, 16, 16
NW = NC * NS
EDGES_PER_TILE = E // NW
CHUNK = 80
N_CHUNKS = EDGES_PER_TILE // CHUNK
DBUF = 5
ABUF = 3
N_PAD = 10240
DEG_PER_TILE = N_PAD // NS
ROWS_PER_TILE = N_PAD // NS

_mesh = plsc.VectorSubcoreMesh(
    core_axis_name="c", subcore_axis_name="s", num_cores=NC, num_subcores=NS
)


@functools.partial(
    pl.kernel,
    out_type=jax.ShapeDtypeStruct((NC * N_PAD,), jnp.float32),
    mesh=_mesh,
    scratch_types=[
        pltpu.VMEM((EDGES_PER_TILE,), jnp.int32),
        pltpu.VMEM((DBUF, CHUNK), jnp.int32),
        pltpu.VMEM((CHUNK,), jnp.float32),
        pltpu.VMEM((DEG_PER_TILE,), jnp.float32),
        pltpu.VMEM_SHARED((N_PAD,), jnp.float32),
        pltpu.SemaphoreType.DMA,
    ] + [pltpu.SemaphoreType.DMA] * DBUF,
)
def _deg_kernel(pk_hbm, out_hbm, pk_v, dst_v, ones_v, zbuf_v, deg_sh, isem,
                *ssem):
    cid = lax.axis_index("c")
    sid = lax.axis_index("s")
    wid = cid * NS + sid

    idx_cp = pltpu.async_copy(
        pk_hbm.at[pl.ds(wid * EDGES_PER_TILE, EDGES_PER_TILE)], pk_v, isem
    )

    zeros16 = jnp.zeros((LANES,), jnp.float32)
    for i in range(CHUNK // LANES):
        ones_v[pl.ds(i * LANES, LANES)] = zeros16 + 1.0

    @pl.loop(0, DEG_PER_TILE // LANES)
    def _(i):
        zbuf_v[pl.ds(i * LANES, LANES)] = zeros16

    pltpu.sync_copy(zbuf_v, deg_sh.at[pl.ds(sid * DEG_PER_TILE, DEG_PER_TILE)])
    plsc.subcore_barrier()
    idx_cp.wait()

    sh16 = jnp.full((LANES,), 16, jnp.int32)

    def _fire(k, b):
        for j in range(CHUNK // LANES):
            w = pk_v[pl.ds(k * CHUNK + j * LANES, LANES)]
            dst_v[b, pl.ds(j * LANES, LANES)] = lax.shift_right_logical(w, sh16)
        pltpu.async_copy(ones_v, deg_sh.at[dst_v.at[b]], ssem[b], add=True)

    def _drain(b):
        pltpu.make_async_copy(ones_v, deg_sh.at[dst_v.at[b]], ssem[b]).wait()

    for b in range(DBUF):
        _fire(b, b)

    @pl.loop(DBUF, N_CHUNKS, step=DBUF)
    def _(i):
        for b in range(DBUF):
            _drain(b)
            _fire(i + b, b)

    for b in range(DBUF):
        _drain(b)

    plsc.subcore_barrier()
    pltpu.sync_copy(
        deg_sh.at[pl.ds(sid * DEG_PER_TILE, DEG_PER_TILE)],
        out_hbm.at[pl.ds(cid * N_PAD + sid * DEG_PER_TILE, DEG_PER_TILE)],
    )


@functools.partial(
    pl.kernel,
    out_type=jax.ShapeDtypeStruct((NC * N_PAD, F), jnp.float32),
    mesh=_mesh,
    scratch_types=[
        pltpu.VMEM((EDGES_PER_TILE,), jnp.int32),
        pltpu.VMEM((ABUF, CHUNK), jnp.int32),
        pltpu.VMEM((ABUF, CHUNK), jnp.int32),
        pltpu.VMEM((ABUF, CHUNK, F), jnp.float32),
        pltpu.VMEM_SHARED((N_PAD, F), jnp.float32),
        pltpu.SemaphoreType.DMA,
    ] + [pltpu.SemaphoreType.DMA] * (2 * ABUF),
)
def _agg_kernel(xs_hbm, pk_hbm, out_hbm, pk_v, src_v, dst_v, rows_v,
                acc_sh, isem0, *sems):
    gsem = sems[:ABUF]
    ssem = sems[ABUF:]
    cid = lax.axis_index("c")
    sid = lax.axis_index("s")
    wid = cid * NS + sid

    pk_cp = pltpu.async_copy(
        pk_hbm.at[pl.ds(wid * EDGES_PER_TILE, EDGES_PER_TILE)], pk_v, isem0
    )

    zeros16 = jnp.zeros((LANES,), jnp.float32)

    @pl.loop(0, CHUNK)
    def _(r):
        for j in range(F // LANES):
            rows_v[0, r, pl.ds(j * LANES, LANES)] = zeros16

    @pl.loop(0, ROWS_PER_TILE // CHUNK)
    def _(i):
        pltpu.sync_copy(
            rows_v.at[0], acc_sh.at[pl.ds(sid * ROWS_PER_TILE + i * CHUNK, CHUNK)]
        )

    plsc.subcore_barrier()
    pk_cp.wait()

    mask16 = jnp.full((LANES,), 0xFFFF, jnp.int32)
    sh16 = jnp.full((LANES,), 16, jnp.int32)

    def _gather(k, b):
        for j in range(CHUNK // LANES):
            w = pk_v[pl.ds(k * CHUNK + j * LANES, LANES)]
            src_v[b, pl.ds(j * LANES, LANES)] = lax.bitwise_and(w, mask16)
            dst_v[b, pl.ds(j * LANES, LANES)] = lax.shift_right_logical(w, sh16)
        pltpu.async_copy(xs_hbm.at[src_v.at[b]], rows_v.at[b], gsem[b])

    def _gwait(b):
        pltpu.make_async_copy(xs_hbm.at[src_v.at[b]], rows_v.at[b],
                              gsem[b]).wait()

    def _scatter(b):
        pltpu.async_copy(rows_v.at[b], acc_sh.at[dst_v.at[b]], ssem[b],
                         add=True)

    def _swait(b):
        pltpu.make_async_copy(rows_v.at[b], acc_sh.at[dst_v.at[b]],
                              ssem[b]).wait()

    for b in range(ABUF):
        _gather(b, b)

    @pl.loop(0, N_CHUNKS, step=ABUF)
    def _(i):
        for b in range(ABUF):
            @pl.when(i + b < N_CHUNKS)
            def _():
                _gwait(b)
                _scatter(b)
        for b in range(ABUF):
            nk = i + b + ABUF

            @pl.when(nk < N_CHUNKS)
            def _():
                _swait(b)
                _gather(nk, b)

    for b in range(ABUF):
        _swait(b)

    plsc.subcore_barrier()
    pltpu.sync_copy(
        acc_sh.at[pl.ds(sid * ROWS_PER_TILE, ROWS_PER_TILE)],
        out_hbm.at[pl.ds(cid * N_PAD + sid * ROWS_PER_TILE, ROWS_PER_TILE)],
    )


_RB = 1000
_GRID = N // _RB

_DN = (((1,), (1,)), ((), ()))


def _dinv_col(deg_ref):
    return lax.rsqrt(deg_ref[:, 0:1] + deg_ref[:, 1:2] + 1.0)


def _scale_body(deg_ref, x_ref, o_ref):
    o_ref[...] = x_ref[...] * _dinv_col(deg_ref)


def _mm_body(deg_ref, p_ref, xs_ref, w1_ref, b1_ref, w2_ref, o_ref):
    dinv = _dinv_col(deg_ref)
    agg = (p_ref[0] + p_ref[1] + xs_ref[...]) * dinv
    h = lax.dot_general(agg, w1_ref[...], _DN,
                        preferred_element_type=jnp.float32)
    h = jnp.maximum(h + b1_ref[...], 0.0)
    hw = lax.dot_general(h, w2_ref[...], _DN,
                         preferred_element_type=jnp.float32)
    o_ref[...] = hw * dinv


def _fin_body(deg_ref, q_ref, hws_ref, b2_ref, o_ref):
    dinv = _dinv_col(deg_ref)
    z = (q_ref[0] + q_ref[1] + hws_ref[...]) * dinv + b2_ref[...]
    m = jnp.max(z, axis=1, keepdims=True)
    e = jnp.exp(z - m)
    s = jnp.sum(e, axis=1, keepdims=True)
    o_ref[...] = z - m - jnp.log(s)


_deg_spec = pl.BlockSpec((_RB, NC), lambda i: (i, 0))
_row_spec = pl.BlockSpec((_RB, F), lambda i: (i, 0))
_pair_spec = pl.BlockSpec((NC, _RB, F), lambda i: (0, i, 0))

_scale_call = pl.pallas_call(
    _scale_body,
    out_shape=jax.ShapeDtypeStruct((N, F), jnp.float32),
    grid=(_GRID,),
    in_specs=[_deg_spec, _row_spec],
    out_specs=_row_spec,
)

_mm_call = pl.pallas_call(
    _mm_body,
    out_shape=jax.ShapeDtypeStruct((N, F), jnp.float32),
    grid=(_GRID,),
    in_specs=[
        _deg_spec,
        _pair_spec,
        _row_spec,
        pl.BlockSpec((256, F), lambda i: (0, 0)),
        pl.BlockSpec((1, 256), lambda i: (0, 0)),
        pl.BlockSpec((F, 256), lambda i: (0, 0)),
    ],
    out_specs=_row_spec,
)

_fin_call = pl.pallas_call(
    _fin_body,
    out_shape=jax.ShapeDtypeStruct((N, F), jnp.float32),
    grid=(_GRID,),
    in_specs=[
        _deg_spec,
        _pair_spec,
        _row_spec,
        pl.BlockSpec((1, F), lambda i: (0, 0)),
    ],
    out_specs=_row_spec,
)


def kernel(x, edge_index, W1, b1, W2, b2):
    packed = lax.shift_left(edge_index[1], 16) | edge_index[0]

    degT = _deg_kernel(packed).reshape(NC, N_PAD).T
    xs = _scale_call(degT, x)
    P = _agg_kernel(xs, packed).reshape(NC, N_PAD, F)
    hws = _mm_call(degT, P, xs, W1, b1.reshape(1, -1), W2)
    Q = _agg_kernel(hws, packed).reshape(NC, N_PAD, F)
    return _fin_call(degT, Q, hws, b2.reshape(1, -1))


# speedup vs baseline: 36.1464x; 1.0188x over previous
import functools

import jax
import jax.numpy as jnp
from jax import lax
from jax.experimental import pallas as pl
from jax.experimental.pallas import tpu as pltpu
from jax.experimental.pallas import tpu_sc as plsc

N = 10000
E = 320000
F = 128
NC, NS, LANES = 2, 16, 16
NW = NC * NS
EDGES_PER_TILE = E // NW
CHUNK = 80
N_CHUNKS = EDGES_PER_TILE // CHUNK
DBUF = 5
ABUF = 3
N_PAD = 10240
DEG_PER_TILE = N_PAD // NS
ROWS_PER_TILE = N_PAD // NS

_mesh = plsc.VectorSubcoreMesh(
    core_axis_name="c", subcore_axis_name="s", num_cores=NC, num_subcores=NS
)


@functools.partial(
    pl.kernel,
    out_type=jax.ShapeDtypeStruct((NC * N_PAD,), jnp.float32),
    mesh=_mesh,
    scratch_types=[
        pltpu.VMEM((EDGES_PER_TILE,), jnp.int32),
        pltpu.VMEM((DBUF, CHUNK), jnp.int32),
        pltpu.VMEM((CHUNK,), jnp.float32),
        pltpu.VMEM((DEG_PER_TILE,), jnp.float32),
        pltpu.VMEM_SHARED((N_PAD,), jnp.float32),
        pltpu.SemaphoreType.DMA,
    ] + [pltpu.SemaphoreType.DMA] * DBUF,
)
def _deg_kernel(ei_hbm, out_hbm, pk_v, dst_v, ones_v, zbuf_v, deg_sh, isem,
                *ssem):
    cid = lax.axis_index("c")
    sid = lax.axis_index("s")
    wid = cid * NS + sid

    idx_cp = pltpu.async_copy(
        ei_hbm.at[pl.ds(E + wid * EDGES_PER_TILE, EDGES_PER_TILE)], pk_v, isem
    )

    zeros16 = jnp.zeros((LANES,), jnp.float32)
    for i in range(CHUNK // LANES):
        ones_v[pl.ds(i * LANES, LANES)] = zeros16 + 1.0

    @pl.loop(0, DEG_PER_TILE // LANES)
    def _(i):
        zbuf_v[pl.ds(i * LANES, LANES)] = zeros16

    pltpu.sync_copy(zbuf_v, deg_sh.at[pl.ds(sid * DEG_PER_TILE, DEG_PER_TILE)])
    plsc.subcore_barrier()
    idx_cp.wait()

    def _fire(k, b):
        for j in range(CHUNK // LANES):
            dst_v[b, pl.ds(j * LANES, LANES)] = (
                pk_v[pl.ds(k * CHUNK + j * LANES, LANES)])
        pltpu.async_copy(ones_v, deg_sh.at[dst_v.at[b]], ssem[b], add=True)

    def _drain(b):
        pltpu.make_async_copy(ones_v, deg_sh.at[dst_v.at[b]], ssem[b]).wait()

    for b in range(DBUF):
        _fire(b, b)

    @pl.loop(DBUF, N_CHUNKS, step=DBUF)
    def _(i):
        for b in range(DBUF):
            _drain(b)
            _fire(i + b, b)

    for b in range(DBUF):
        _drain(b)

    plsc.subcore_barrier()
    pltpu.sync_copy(
        deg_sh.at[pl.ds(sid * DEG_PER_TILE, DEG_PER_TILE)],
        out_hbm.at[pl.ds(cid * N_PAD + sid * DEG_PER_TILE, DEG_PER_TILE)],
    )


@functools.partial(
    pl.kernel,
    out_type=jax.ShapeDtypeStruct((NC * N_PAD, F), jnp.float32),
    mesh=_mesh,
    scratch_types=[
        pltpu.VMEM((ABUF, CHUNK), jnp.int32),
        pltpu.VMEM((ABUF, CHUNK), jnp.int32),
        pltpu.VMEM((ABUF, CHUNK, F), jnp.float32),
        pltpu.VMEM_SHARED((N_PAD, F), jnp.float32),
    ] + [pltpu.SemaphoreType.DMA] * (4 * ABUF),
)
def _agg_kernel(xs_hbm, ei_hbm, out_hbm, src_v, dst_v, rows_v, acc_sh, *sems):
    issem = sems[:ABUF]
    idsem = sems[ABUF:2 * ABUF]
    gsem = sems[2 * ABUF:3 * ABUF]
    ssem = sems[3 * ABUF:]
    cid = lax.axis_index("c")
    sid = lax.axis_index("s")
    wid = cid * NS + sid
    base = wid * EDGES_PER_TILE

    def _isfire(k, b):
        pltpu.async_copy(ei_hbm.at[pl.ds(base + k * CHUNK, CHUNK)],
                         src_v.at[b], issem[b])

    def _iswait(b):
        pltpu.make_async_copy(ei_hbm.at[pl.ds(base, CHUNK)],
                              src_v.at[b], issem[b]).wait()

    def _idfire(k, b):
        pltpu.async_copy(ei_hbm.at[pl.ds(E + base + k * CHUNK, CHUNK)],
                         dst_v.at[b], idsem[b])

    def _idwait(b):
        pltpu.make_async_copy(ei_hbm.at[pl.ds(E + base, CHUNK)],
                              dst_v.at[b], idsem[b]).wait()

    for b in range(ABUF):
        _isfire(b, b)
        _idfire(b, b)

    zeros16 = jnp.zeros((LANES,), jnp.float32)

    @pl.loop(0, CHUNK)
    def _(r):
        for j in range(F // LANES):
            rows_v[0, r, pl.ds(j * LANES, LANES)] = zeros16

    @pl.loop(0, ROWS_PER_TILE // CHUNK)
    def _(i):
        pltpu.sync_copy(
            rows_v.at[0], acc_sh.at[pl.ds(sid * ROWS_PER_TILE + i * CHUNK, CHUNK)]
        )

    plsc.subcore_barrier()

    def _gather(b):
        _iswait(b)
        pltpu.async_copy(xs_hbm.at[src_v.at[b]], rows_v.at[b], gsem[b])

    def _gwait(b):
        pltpu.make_async_copy(xs_hbm.at[src_v.at[b]], rows_v.at[b],
                              gsem[b]).wait()

    def _scatter(b):
        _idwait(b)
        pltpu.async_copy(rows_v.at[b], acc_sh.at[dst_v.at[b]], ssem[b],
                         add=True)

    def _swait(b):
        pltpu.make_async_copy(rows_v.at[b], acc_sh.at[dst_v.at[b]],
                              ssem[b]).wait()

    for b in range(ABUF):
        _gather(b)

    @pl.loop(0, N_CHUNKS, step=ABUF)
    def _(i):
        for b in range(ABUF):
            @pl.when(i + b < N_CHUNKS)
            def _():
                _gwait(b)
                @pl.when(i + b + ABUF < N_CHUNKS)
                def _():
                    _isfire(i + b + ABUF, b)
                _scatter(b)
        for b in range(ABUF):
            nk = i + b + ABUF

            @pl.when(nk < N_CHUNKS)
            def _():
                _swait(b)
                _idfire(nk, b)
                _gather(b)

    for b in range(ABUF):
        _swait(b)

    plsc.subcore_barrier()
    pltpu.sync_copy(
        acc_sh.at[pl.ds(sid * ROWS_PER_TILE, ROWS_PER_TILE)],
        out_hbm.at[pl.ds(cid * N_PAD + sid * ROWS_PER_TILE, ROWS_PER_TILE)],
    )


_RB = 1000
_GRID = N // _RB

_DN = (((1,), (1,)), ((), ()))


def _dinv_col(deg_ref):
    return lax.rsqrt(deg_ref[:, 0:1] + deg_ref[:, 1:2] + 1.0)


def _scale_body(deg_ref, x_ref, o_ref):
    o_ref[...] = x_ref[...] * _dinv_col(deg_ref)


def _mm_body(deg_ref, p_ref, xs_ref, w1_ref, b1_ref, w2_ref, o_ref):
    dinv = _dinv_col(deg_ref)
    agg = (p_ref[0] + p_ref[1] + xs_ref[...]) * dinv
    h = lax.dot_general(agg, w1_ref[...], _DN,
                        preferred_element_type=jnp.float32)
    h = jnp.maximum(h + b1_ref[...], 0.0)
    hw = lax.dot_general(h, w2_ref[...], _DN,
                         preferred_element_type=jnp.float32)
    o_ref[...] = hw * dinv


def _fin_body(deg_ref, q_ref, hws_ref, b2_ref, o_ref):
    dinv = _dinv_col(deg_ref)
    z = (q_ref[0] + q_ref[1] + hws_ref[...]) * dinv + b2_ref[...]
    m = jnp.max(z, axis=1, keepdims=True)
    e = jnp.exp(z - m)
    s = jnp.sum(e, axis=1, keepdims=True)
    o_ref[...] = z - m - jnp.log(s)


_deg_spec = pl.BlockSpec((_RB, NC), lambda i: (i, 0))
_row_spec = pl.BlockSpec((_RB, F), lambda i: (i, 0))
_pair_spec = pl.BlockSpec((NC, _RB, F), lambda i: (0, i, 0))

_scale_call = pl.pallas_call(
    _scale_body,
    out_shape=jax.ShapeDtypeStruct((N, F), jnp.float32),
    grid=(_GRID,),
    in_specs=[_deg_spec, _row_spec],
    out_specs=_row_spec,
)

_mm_call = pl.pallas_call(
    _mm_body,
    out_shape=jax.ShapeDtypeStruct((N, F), jnp.float32),
    grid=(_GRID,),
    in_specs=[
        _deg_spec,
        _pair_spec,
        _row_spec,
        pl.BlockSpec((256, F), lambda i: (0, 0)),
        pl.BlockSpec((1, 256), lambda i: (0, 0)),
        pl.BlockSpec((F, 256), lambda i: (0, 0)),
    ],
    out_specs=_row_spec,
)

_fin_call = pl.pallas_call(
    _fin_body,
    out_shape=jax.ShapeDtypeStruct((N, F), jnp.float32),
    grid=(_GRID,),
    in_specs=[
        _deg_spec,
        _pair_spec,
        _row_spec,
        pl.BlockSpec((1, F), lambda i: (0, 0)),
    ],
    out_specs=_row_spec,
)


def kernel(x, edge_index, W1, b1, W2, b2):
    ei_flat = edge_index.reshape(2 * E)

    degT = _deg_kernel(ei_flat).reshape(NC, N_PAD).T
    xs = _scale_call(degT, x)
    P = _agg_kernel(xs, ei_flat).reshape(NC, N_PAD, F)
    hws = _mm_call(degT, P, xs, W1, b1.reshape(1, -1), W2)
    Q = _agg_kernel(hws, ei_flat).reshape(NC, N_PAD, F)
    return _fin_call(degT, Q, hws, b2.reshape(1, -1))


# speedup vs baseline: 38.3552x; 1.0611x over previous
import functools

import jax
import jax.numpy as jnp
from jax import lax
from jax.experimental import pallas as pl
from jax.experimental.pallas import tpu as pltpu
from jax.experimental.pallas import tpu_sc as plsc

N = 10000
E = 320000
F = 128
NC, NS, LANES = 2, 16, 16
NW = NC * NS
EDGES_PER_TILE = E // NW
CHUNK = 80
N_CHUNKS = EDGES_PER_TILE // CHUNK
DBUF = 5
ABUF = 4
ACHUNK = 64
AG_CHUNKS = E // ACHUNK
N_PAD = 10240
DEG_PER_TILE = N_PAD // NS
ROWS_PER_TILE = N_PAD // NS

_mesh = plsc.VectorSubcoreMesh(
    core_axis_name="c", subcore_axis_name="s", num_cores=NC, num_subcores=NS
)


@functools.partial(
    pl.kernel,
    out_type=jax.ShapeDtypeStruct((NC * N_PAD,), jnp.float32),
    mesh=_mesh,
    scratch_types=[
        pltpu.VMEM((EDGES_PER_TILE,), jnp.int32),
        pltpu.VMEM((DBUF, CHUNK), jnp.int32),
        pltpu.VMEM((CHUNK,), jnp.float32),
        pltpu.VMEM((DEG_PER_TILE,), jnp.float32),
        pltpu.VMEM_SHARED((N_PAD,), jnp.float32),
        pltpu.SemaphoreType.DMA,
    ] + [pltpu.SemaphoreType.DMA] * DBUF,
)
def _deg_kernel(ei_hbm, out_hbm, pk_v, dst_v, ones_v, zbuf_v, deg_sh, isem,
                *ssem):
    cid = lax.axis_index("c")
    sid = lax.axis_index("s")
    wid = cid * NS + sid

    idx_cp = pltpu.async_copy(
        ei_hbm.at[pl.ds(E + wid * EDGES_PER_TILE, EDGES_PER_TILE)], pk_v, isem
    )

    zeros16 = jnp.zeros((LANES,), jnp.float32)
    for i in range(CHUNK // LANES):
        ones_v[pl.ds(i * LANES, LANES)] = zeros16 + 1.0

    @pl.loop(0, DEG_PER_TILE // LANES)
    def _(i):
        zbuf_v[pl.ds(i * LANES, LANES)] = zeros16

    pltpu.sync_copy(zbuf_v, deg_sh.at[pl.ds(sid * DEG_PER_TILE, DEG_PER_TILE)])
    plsc.subcore_barrier()
    idx_cp.wait()

    def _fire(k, b):
        for j in range(CHUNK // LANES):
            dst_v[b, pl.ds(j * LANES, LANES)] = (
                pk_v[pl.ds(k * CHUNK + j * LANES, LANES)])
        pltpu.async_copy(ones_v, deg_sh.at[dst_v.at[b]], ssem[b], add=True)

    def _drain(b):
        pltpu.make_async_copy(ones_v, deg_sh.at[dst_v.at[b]], ssem[b]).wait()

    for b in range(DBUF):
        _fire(b, b)

    @pl.loop(DBUF, N_CHUNKS, step=DBUF)
    def _(i):
        for b in range(DBUF):
            _drain(b)
            _fire(i + b, b)

    for b in range(DBUF):
        _drain(b)

    plsc.subcore_barrier()
    pltpu.sync_copy(
        deg_sh.at[pl.ds(sid * DEG_PER_TILE, DEG_PER_TILE)],
        out_hbm.at[pl.ds(cid * N_PAD + sid * DEG_PER_TILE, DEG_PER_TILE)],
    )


@functools.partial(
    pl.kernel,
    out_type=jax.ShapeDtypeStruct((NC * N_PAD, F), jnp.float32),
    mesh=_mesh,
    scratch_types=[
        pltpu.VMEM((ABUF, ACHUNK), jnp.int32),
        pltpu.VMEM((ABUF, ACHUNK), jnp.int32),
        pltpu.VMEM((ABUF, ACHUNK, F), jnp.float32),
        pltpu.VMEM_SHARED((N_PAD, F), jnp.float32),
    ] + [pltpu.SemaphoreType.DMA] * (4 * ABUF),
)
def _agg_kernel(xs_hbm, ei_hbm, out_hbm, src_v, dst_v, rows_v, acc_sh, *sems):
    issem = sems[:ABUF]
    idsem = sems[ABUF:2 * ABUF]
    gsem = sems[2 * ABUF:3 * ABUF]
    ssem = sems[3 * ABUF:]
    cid = lax.axis_index("c")
    sid = lax.axis_index("s")
    wid = cid * NS + sid
    my_n = jnp.where(wid < AG_CHUNKS % NW,
                     AG_CHUNKS // NW + 1, AG_CHUNKS // NW)

    def _isfire(k, b):
        off = (wid + k * NW) * ACHUNK
        pltpu.async_copy(ei_hbm.at[pl.ds(off, ACHUNK)], src_v.at[b], issem[b])

    def _iswait(b):
        pltpu.make_async_copy(ei_hbm.at[pl.ds(0, ACHUNK)],
                              src_v.at[b], issem[b]).wait()

    def _idfire(k, b):
        off = E + (wid + k * NW) * ACHUNK
        pltpu.async_copy(ei_hbm.at[pl.ds(off, ACHUNK)], dst_v.at[b], idsem[b])

    def _idwait(b):
        pltpu.make_async_copy(ei_hbm.at[pl.ds(0, ACHUNK)],
                              dst_v.at[b], idsem[b]).wait()

    for b in range(ABUF):
        _isfire(b, b)
        _idfire(b, b)

    zeros16 = jnp.zeros((LANES,), jnp.float32)

    @pl.loop(0, ACHUNK)
    def _(r):
        for j in range(F // LANES):
            rows_v[0, r, pl.ds(j * LANES, LANES)] = zeros16

    @pl.loop(0, ROWS_PER_TILE // ACHUNK)
    def _(i):
        pltpu.sync_copy(
            rows_v.at[0],
            acc_sh.at[pl.ds(sid * ROWS_PER_TILE + i * ACHUNK, ACHUNK)]
        )

    plsc.subcore_barrier()

    def _gather(b):
        _iswait(b)
        pltpu.async_copy(xs_hbm.at[src_v.at[b]], rows_v.at[b], gsem[b])

    def _gwait(b):
        pltpu.make_async_copy(xs_hbm.at[src_v.at[b]], rows_v.at[b],
                              gsem[b]).wait()

    def _scatter(b):
        _idwait(b)
        pltpu.async_copy(rows_v.at[b], acc_sh.at[dst_v.at[b]], ssem[b],
                         add=True)

    def _swait(b):
        pltpu.make_async_copy(rows_v.at[b], acc_sh.at[dst_v.at[b]],
                              ssem[b]).wait()

    for b in range(ABUF):
        _gather(b)

    @pl.loop(0, my_n, step=ABUF)
    def _(i):
        for b in range(ABUF):
            @pl.when(i + b < my_n)
            def _():
                _gwait(b)
                @pl.when(i + b + ABUF < my_n)
                def _():
                    _isfire(i + b + ABUF, b)
                _scatter(b)
        for b in range(ABUF):
            nk = i + b + ABUF

            @pl.when(nk < my_n)
            def _():
                _swait(b)
                _idfire(nk, b)
                _gather(b)

    for b in range(ABUF):
        _swait(b)

    plsc.subcore_barrier()
    pltpu.sync_copy(
        acc_sh.at[pl.ds(sid * ROWS_PER_TILE, ROWS_PER_TILE)],
        out_hbm.at[pl.ds(cid * N_PAD + sid * ROWS_PER_TILE, ROWS_PER_TILE)],
    )


_RB = 1000
_GRID = N // _RB

_DN = (((1,), (1,)), ((), ()))


def _dinv_col(deg_ref):
    return lax.rsqrt(deg_ref[:, 0:1] + deg_ref[:, 1:2] + 1.0)


def _scale_body(deg_ref, x_ref, o_ref):
    o_ref[...] = x_ref[...] * _dinv_col(deg_ref)


def _mm_body(deg_ref, p_ref, xs_ref, w1_ref, b1_ref, w2_ref, o_ref):
    dinv = _dinv_col(deg_ref)
    agg = (p_ref[0] + p_ref[1] + xs_ref[...]) * dinv
    h = lax.dot_general(agg, w1_ref[...], _DN,
                        preferred_element_type=jnp.float32)
    h = jnp.maximum(h + b1_ref[...], 0.0)
    hw = lax.dot_general(h, w2_ref[...], _DN,
                         preferred_element_type=jnp.float32)
    o_ref[...] = hw * dinv


def _fin_body(deg_ref, q_ref, hws_ref, b2_ref, o_ref):
    dinv = _dinv_col(deg_ref)
    z = (q_ref[0] + q_ref[1] + hws_ref[...]) * dinv + b2_ref[...]
    m = jnp.max(z, axis=1, keepdims=True)
    e = jnp.exp(z - m)
    s = jnp.sum(e, axis=1, keepdims=True)
    o_ref[...] = z - m - jnp.log(s)


_deg_spec = pl.BlockSpec((_RB, NC), lambda i: (i, 0))
_row_spec = pl.BlockSpec((_RB, F), lambda i: (i, 0))
_pair_spec = pl.BlockSpec((NC, _RB, F), lambda i: (0, i, 0))

_scale_call = pl.pallas_call(
    _scale_body,
    out_shape=jax.ShapeDtypeStruct((N, F), jnp.float32),
    grid=(_GRID,),
    in_specs=[_deg_spec, _row_spec],
    out_specs=_row_spec,
)

_mm_call = pl.pallas_call(
    _mm_body,
    out_shape=jax.ShapeDtypeStruct((N, F), jnp.float32),
    grid=(_GRID,),
    in_specs=[
        _deg_spec,
        _pair_spec,
        _row_spec,
        pl.BlockSpec((256, F), lambda i: (0, 0)),
        pl.BlockSpec((1, 256), lambda i: (0, 0)),
        pl.BlockSpec((F, 256), lambda i: (0, 0)),
    ],
    out_specs=_row_spec,
)

_fin_call = pl.pallas_call(
    _fin_body,
    out_shape=jax.ShapeDtypeStruct((N, F), jnp.float32),
    grid=(_GRID,),
    in_specs=[
        _deg_spec,
        _pair_spec,
        _row_spec,
        pl.BlockSpec((1, F), lambda i: (0, 0)),
    ],
    out_specs=_row_spec,
)


def kernel(x, edge_index, W1, b1, W2, b2):
    ei_flat = edge_index.reshape(2 * E)

    degT = _deg_kernel(ei_flat).reshape(NC, N_PAD).T
    xs = _scale_call(degT, x)
    P = _agg_kernel(xs, ei_flat).reshape(NC, N_PAD, F)
    hws = _mm_call(degT, P, xs, W1, b1.reshape(1, -1), W2)
    Q = _agg_kernel(hws, ei_flat).reshape(NC, N_PAD, F)
    return _fin_call(degT, Q, hws, b2.reshape(1, -1))
